# Initial kernel scaffold; baseline (speedup 1.0000x reference)
import jax
import jax.numpy as jnp
from jax.experimental import pallas as pl


def kernel(pattern, pattern_gt, mask):
    raise NotImplementedError("write your pallas kernel here")


# speedup vs baseline: 34.8882x; 34.8882x over previous
import functools

import jax
import jax.numpy as jnp
from jax import lax
from jax.experimental import pallas as pl
from jax.experimental.pallas import tpu as pltpu
from jax.experimental.pallas import tpu_sc as plsc

C = 96
H = 384
W = 384
HW = H * W
NC = 2
NS = 16
NW = NC * NS
CPW = C // NW
CHUNK = 73728
NCHUNK = HW // CHUNK
NVREG = CHUNK // 16
TOPK = 10
NCAND = 16 * TOPK


def _sc_top10_cands(x_flat):
    mesh = plsc.VectorSubcoreMesh(
        core_axis_name="c", subcore_axis_name="s", num_cores=NC, num_subcores=NS
    )

    @functools.partial(
        pl.kernel,
        out_type=jax.ShapeDtypeStruct((C, NCAND), jnp.float32),
        mesh=mesh,
        scratch_types=[
            pltpu.VMEM((CHUNK,), jnp.float32),
            pltpu.VMEM((NCAND,), jnp.float32),
        ],
    )
    def k(x_hbm, out_hbm, buf, obuf):
        wid = lax.axis_index("s") * NC + lax.axis_index("c")
        neg = jnp.full((16,), -jnp.inf, jnp.float32)
        for ci in range(CPW):
            ch = wid * CPW + ci
            rs = (neg,) * TOPK
            for ck in range(NCHUNK):
                pltpu.sync_copy(
                    x_hbm.at[pl.ds(ch * HW + ck * CHUNK, CHUNK)], buf
                )

                def body(i, carry):
                    v = buf[pl.ds(i * 16, 16)]
                    out = []
                    for r in carry:
                        hi = jnp.maximum(r, v)
                        v = jnp.minimum(r, v)
                        out.append(hi)
                    return tuple(out)

                rs = lax.fori_loop(0, NVREG, body, rs)
            for kk in range(TOPK):
                obuf[pl.ds(16 * kk, 16)] = rs[kk]
            pltpu.sync_copy(obuf, out_hbm.at[ch])

    return k(x_flat)


def _tc_loss(x3, gt, msk, cands):
    def body(c_ref, x_ref, gt_ref, m_ref, o_ref):
        i = pl.program_id(0)

        @pl.when(i == 0)
        def _():
            o_ref[0, 0] = 0.0

        c = c_ref[0]
        pos = lax.broadcasted_iota(jnp.int32, (1, NCAND), 1)
        t10 = 0.0
        for _ in range(TOPK):
            m = jnp.max(c)
            t10 = t10 + m
            first = jnp.min(jnp.where(c == m, pos, NCAND))
            c = jnp.where(pos == first, -jnp.inf, c)

        x = x_ref[0]
        mean = jnp.sum(x) * (1.0 / HW)
        d = t10 * (1.0 / TOPK) - mean
        denom = jnp.where(d < 1e-20, d + 1e-19, d)
        e = ((x - mean) * (1.0 / denom) - gt_ref[...]) * m_ref[...]
        o_ref[0, 0] += jnp.sum(e * e) * (1.0 / HW)

    return pl.pallas_call(
        body,
        grid=(C,),
        in_specs=[
            pl.BlockSpec((1, 1, NCAND), lambda i: (i, 0, 0)),
            pl.BlockSpec((1, H, W), lambda i: (i, 0, 0)),
            pl.BlockSpec((H, W), lambda i: (0, 0)),
            pl.BlockSpec((H, W), lambda i: (0, 0)),
        ],
        out_specs=pl.BlockSpec((1, 1), lambda i: (0, 0), memory_space=pltpu.SMEM),
        out_shape=jax.ShapeDtypeStruct((1, 1), jnp.float32),
    )(cands, x3, gt, msk)


def kernel(pattern, pattern_gt, mask):
    x3 = pattern.reshape(C, H, W)
    cands = _sc_top10_cands(pattern.reshape(C * HW))
    loss = _tc_loss(x3, pattern_gt, mask, cands.reshape(C, 1, NCAND))
    return loss.reshape(1)


# speedup vs baseline: 86.8623x; 2.4897x over previous
import functools

import jax
import jax.numpy as jnp
from jax import lax
from jax.experimental import pallas as pl
from jax.experimental.pallas import tpu as pltpu
from jax.experimental.pallas import tpu_sc as plsc

C = 96
H = 384
W = 384
HW = H * W
NC = 2
NS = 16
NW = NC * NS
CPW = C // NW
CHUNK = 73728
NCHUNK = HW // CHUNK
NVREG = CHUNK // 16
TOPK = 10
NCAND = 16 * TOPK
CB = 4


def _sc_top10_cands(x_flat):
    mesh = plsc.VectorSubcoreMesh(
        core_axis_name="c", subcore_axis_name="s", num_cores=NC, num_subcores=NS
    )

    @functools.partial(
        pl.kernel,
        out_type=jax.ShapeDtypeStruct((C, NCAND), jnp.float32),
        mesh=mesh,
        scratch_types=[
            pltpu.VMEM((CHUNK,), jnp.float32),
            pltpu.VMEM((NCAND,), jnp.float32),
        ],
    )
    def k(x_hbm, out_hbm, buf, obuf):
        wid = lax.axis_index("s") * NC + lax.axis_index("c")
        neg = jnp.full((16,), -jnp.inf, jnp.float32)
        for ci in range(CPW):
            ch = wid * CPW + ci
            rs = (neg,) * TOPK
            for ck in range(NCHUNK):
                pltpu.sync_copy(
                    x_hbm.at[pl.ds(ch * HW + ck * CHUNK, CHUNK)], buf
                )

                def body(i, carry):
                    v = buf[pl.ds(i * 16, 16)]
                    out = []
                    for r in carry:
                        hi = jnp.maximum(r, v)
                        v = jnp.minimum(r, v)
                        out.append(hi)
                    return tuple(out)

                rs = lax.fori_loop(0, NVREG, body, rs)
            for kk in range(TOPK):
                obuf[pl.ds(16 * kk, 16)] = rs[kk]
            pltpu.sync_copy(obuf, out_hbm.at[ch])

    return k(x_flat)


def _tc_sums(x3, gt, msk):
    def body(x_ref, gt_ref, m_ref, o_ref):
        mm = m_ref[...]
        w2 = mm * mm
        wg = w2 * gt_ref[...]
        rows = []
        for c in range(CB):
            x = x_ref[c]
            t = x * w2
            a = jnp.sum(t * x, keepdims=True).reshape(1, 1)
            b = jnp.sum(t, keepdims=True).reshape(1, 1)
            cg = jnp.sum(x * wg, keepdims=True).reshape(1, 1)
            s = jnp.sum(x, keepdims=True).reshape(1, 1)
            rows.append(jnp.concatenate([a, b, cg, s], axis=1))
        o_ref[0] = jnp.concatenate(rows, axis=0)

    return pl.pallas_call(
        body,
        grid=(C // CB,),
        in_specs=[
            pl.BlockSpec((CB, H, W), lambda i: (i, 0, 0)),
            pl.BlockSpec((H, W), lambda i: (0, 0)),
            pl.BlockSpec((H, W), lambda i: (0, 0)),
        ],
        out_specs=pl.BlockSpec((1, CB, 4), lambda i: (i, 0, 0)),
        out_shape=jax.ShapeDtypeStruct((C // CB, CB, 4), jnp.float32),
    )(x3, gt, msk)


def _tc_combine(cands, abcs, gt, msk):
    def body(c_ref, ab_ref, gt_ref, m_ref, o_ref):
        mm = m_ref[...]
        w2 = mm * mm
        wg = w2 * gt_ref[...]
        m2c = jnp.sum(w2)
        gc = jnp.sum(wg)
        g2c = jnp.sum(wg * gt_ref[...])

        c = c_ref[...]
        pos = lax.broadcasted_iota(jnp.int32, (C, NCAND), 1)
        t10 = jnp.zeros((C, 1), jnp.float32)
        for _ in range(TOPK):
            mx = jnp.max(c, axis=1, keepdims=True)
            t10 = t10 + mx
            first = jnp.min(
                jnp.where(c == mx, pos, NCAND), axis=1, keepdims=True
            )
            c = jnp.where(pos == first, -jnp.inf, c)

        ab = ab_ref[...]
        a = ab[:, 0:1]
        b = ab[:, 1:2]
        cg = ab[:, 2:3]
        s = ab[:, 3:4]
        m = s * (1.0 / HW)
        d = t10 * (1.0 / TOPK) - m
        denom = jnp.where(d < 1e-20, d + 1e-19, d)
        inv = 1.0 / denom
        li = (
            inv * inv * (a - 2.0 * m * b + m * m * m2c)
            - 2.0 * inv * (cg - m * gc)
            + g2c
        ) * (1.0 / HW)
        o_ref[0, 0] = jnp.sum(li)

    return pl.pallas_call(
        body,
        in_specs=[
            pl.BlockSpec((C, NCAND), lambda: (0, 0)),
            pl.BlockSpec((C, 4), lambda: (0, 0)),
            pl.BlockSpec((H, W), lambda: (0, 0)),
            pl.BlockSpec((H, W), lambda: (0, 0)),
        ],
        out_specs=pl.BlockSpec((1, 1), lambda: (0, 0), memory_space=pltpu.SMEM),
        out_shape=jax.ShapeDtypeStruct((1, 1), jnp.float32),
    )(cands, abcs, gt, msk)


def kernel(pattern, pattern_gt, mask):
    x3 = pattern.reshape(C, H, W)
    cands = _sc_top10_cands(pattern.reshape(C * HW))
    abcs = _tc_sums(x3, pattern_gt, mask)
    loss = _tc_combine(cands, abcs.reshape(C, 4), pattern_gt, mask)
    return loss.reshape(1)
